# baseline (device time: 337315 ns/iter reference)
import jax
import jax.numpy as jnp
from jax import lax
from jax.experimental import pallas as pl
from jax.experimental.pallas import tpu as pltpu

N_DEV = 4
HQ = 8
DH = 128
SQ = 256
SKV = 4096
BLK = 64
SCALE = 0.08838834764831843
NEG = -1e9
M_INIT = -1e30


def _attn_ring(q_in, k_in, v_in):

    def body(q_ref, k_ref, v_ref, out_ref,
             q_buf, acc_buf, stat_buf,
             q_send, q_recv, acc_send, acc_recv, stat_send, stat_recv):
        me = lax.axis_index("i")
        left = (me + N_DEV - 1) % N_DEV
        right = (me + 1) % N_DEV

        barrier = pltpu.get_barrier_semaphore()
        for nbr in (left, right):
            pl.semaphore_signal(barrier, inc=1, device_id=(nbr,),
                                device_id_type=pl.DeviceIdType.MESH)
        pl.semaphore_wait(barrier, 2)

        q_buf[0] = q_ref[...]
        acc_buf[0] = jnp.zeros((HQ, SQ, DH), jnp.float32)
        stat_buf[0, 0] = jnp.full((SQ, HQ), M_INIT, jnp.float32)
        stat_buf[0, 1] = jnp.zeros((SQ, HQ), jnp.float32)

        kb = (me * SKV
              + lax.broadcasted_iota(jnp.int32, (1, SKV), 1)) // BLK
        head_ids = lax.broadcasted_iota(jnp.int32, (1, HQ), 1)

        in_flight = []

        def recv_wait(buf, dst_slot, ssem, rsem, hop):
            d = pltpu.make_async_remote_copy(
                src_ref=buf.at[dst_slot], dst_ref=buf.at[dst_slot],
                send_sem=ssem.at[hop], recv_sem=rsem.at[hop],
                device_id=(left,), device_id_type=pl.DeviceIdType.MESH)
            d.wait_recv()

        def send(buf, src_slot, dst_slot, ssem, rsem, hop):
            d = pltpu.make_async_remote_copy(
                src_ref=buf.at[src_slot], dst_ref=buf.at[dst_slot],
                send_sem=ssem.at[hop], recv_sem=rsem.at[hop],
                device_id=(right,), device_id_type=pl.DeviceIdType.MESH)
            d.start()
            in_flight.append(d)

        for r in range(N_DEV):
            if r > 0:
                recv_wait(q_buf, r, q_send, q_recv, r - 1)
            if r < N_DEV - 1:
                send(q_buf, r, r + 1, q_send, q_recv, r)
            if r > 0:
                recv_wait(acc_buf, r, acc_send, acc_recv, r - 1)
                recv_wait(stat_buf, r, stat_send, stat_recv, r - 1)

            q_idx = (me + N_DEV - r) % N_DEV
            qb = (q_idx * SQ
                  + lax.broadcasted_iota(jnp.int32, (SQ, 1), 0)) // BLK
            mask = (qb == kb) | (kb == 0) | ((qb + kb) % 3 == 0)

            def round_h(h, _, r=r, mask=mask):
                oh = (head_ids == h).astype(jnp.float32)
                m_all = stat_buf[r, 0]
                l_all = stat_buf[r, 1]
                m_old = jnp.sum(m_all * oh, axis=1, keepdims=True)
                l_old = jnp.sum(l_all * oh, axis=1, keepdims=True)

                qh = q_buf[r, h].astype(jnp.bfloat16)
                kh = k_ref[0, :, h, :].astype(jnp.bfloat16)
                s = lax.dot_general(
                    qh, kh, (((1,), (1,)), ((), ())),
                    preferred_element_type=jnp.float32) * SCALE
                s = jnp.where(mask, s, NEG)
                m_new = jnp.maximum(m_old,
                                    jnp.max(s, axis=1, keepdims=True))
                p = jnp.exp(s - m_new)
                alpha = jnp.exp(m_old - m_new)
                l_new = l_old * alpha + jnp.sum(p, axis=1, keepdims=True)
                vh = v_ref[0, :, h, :].astype(jnp.bfloat16)
                pv = lax.dot_general(p.astype(jnp.bfloat16), vh,
                                     (((1,), (0,)), ((), ())),
                                     preferred_element_type=jnp.float32)
                acc_buf[r, h] = acc_buf[r, h] * alpha + pv
                stat_buf[r, 0] = m_all * (1.0 - oh) + m_new * oh
                stat_buf[r, 1] = l_all * (1.0 - oh) + l_new * oh
                return _
            lax.fori_loop(0, HQ, round_h, None)

            if r < N_DEV - 1:
                send(acc_buf, r, r + 1, acc_send, acc_recv, r)
                send(stat_buf, r, r + 1, stat_send, stat_recv, r)

        def fin_h(h, _):
            oh = (head_ids == h).astype(jnp.float32)
            l_fin = jnp.sum(stat_buf[N_DEV - 1, 1] * oh, axis=1,
                            keepdims=True)
            acc_buf[N_DEV - 1, h] = acc_buf[N_DEV - 1, h] / l_fin
            return _
        lax.fori_loop(0, HQ, fin_h, None)
        send(acc_buf, N_DEV - 1, N_DEV, acc_send, acc_recv, N_DEV - 1)
        recv_wait(acc_buf, N_DEV, acc_send, acc_recv, N_DEV - 1)

        out_ref[...] = acc_buf[N_DEV]

        for d in in_flight:
            d.wait_send()

    return pl.pallas_call(
        body,
        out_shape=jax.ShapeDtypeStruct((HQ, SQ, DH), jnp.float32),
        in_specs=[pl.BlockSpec(memory_space=pltpu.VMEM)] * 3,
        out_specs=pl.BlockSpec(memory_space=pltpu.VMEM),
        scratch_shapes=[
            pltpu.VMEM((N_DEV, HQ, SQ, DH), jnp.float32),
            pltpu.VMEM((N_DEV + 1, HQ, SQ, DH), jnp.float32),
            pltpu.VMEM((N_DEV, 2, SQ, HQ), jnp.float32),
            pltpu.SemaphoreType.DMA((N_DEV - 1,)),
            pltpu.SemaphoreType.DMA((N_DEV - 1,)),
            pltpu.SemaphoreType.DMA((N_DEV,)),
            pltpu.SemaphoreType.DMA((N_DEV,)),
            pltpu.SemaphoreType.DMA((N_DEV - 1,)),
            pltpu.SemaphoreType.DMA((N_DEV - 1,)),
        ],
        compiler_params=pltpu.CompilerParams(
            collective_id=0, vmem_limit_bytes=100 * 1024 * 1024),
    )(q_in, k_in, v_in)


def kernel(x, Wq, K_ext, V_ext, Wo):
    q = (x[0] @ Wq).reshape(SQ, HQ, DH).transpose(1, 0, 2)
    ctx = _attn_ring(q, K_ext, V_ext)
    out = ctx.transpose(1, 0, 2).reshape(SQ, HQ * DH) @ Wo
    return out[None]


# device time: 187672 ns/iter; 1.7974x vs baseline; 1.7974x over previous
import jax
import jax.numpy as jnp
from jax import lax
from jax.experimental import pallas as pl
from jax.experimental.pallas import tpu as pltpu

N_DEV = 4
HQ = 8
DH = 128
SQ = 256
SKV = 4096
BLK = 64
SCALE = 0.08838834764831843
NEG = -1e9


def _attn_ring(q_in, k_in, v_in):

    def body(q_ref, k_ref, v_ref, out_ref,
             q_buf, psend_buf, precv_buf, lst_buf, lrecv_buf,
             q_send, q_recv, p_send, p_recv, l_send, l_recv):
        me = lax.axis_index("i")
        left = (me + N_DEV - 1) % N_DEV
        right = (me + 1) % N_DEV

        barrier = pltpu.get_barrier_semaphore()
        for k in range(1, N_DEV):
            pl.semaphore_signal(barrier, inc=1,
                                device_id=((me + k) % N_DEV,),
                                device_id_type=pl.DeviceIdType.MESH)
        pl.semaphore_wait(barrier, N_DEV - 1)

        q_buf[0] = q_ref[...]

        kb = (me * SKV
              + lax.broadcasted_iota(jnp.int32, (1, SKV), 1)) // BLK
        head_ids = lax.broadcasted_iota(jnp.int32, (1, HQ), 1)

        in_flight = []

        def rdma(src, dst, ssem, rsem, target):
            return pltpu.make_async_remote_copy(
                src_ref=src, dst_ref=dst, send_sem=ssem, recv_sem=rsem,
                device_id=(target,), device_id_type=pl.DeviceIdType.MESH)

        for r in range(N_DEV):
            if r > 0:
                rdma(q_buf.at[r], q_buf.at[r],
                     q_send.at[r - 1], q_recv.at[r - 1], left).wait_recv()
            if r < N_DEV - 1:
                d = rdma(q_buf.at[r], q_buf.at[r + 1],
                         q_send.at[r], q_recv.at[r], right)
                d.start()
                in_flight.append(d)

            q_idx = (me + N_DEV - r) % N_DEV
            qb = (q_idx * SQ
                  + lax.broadcasted_iota(jnp.int32, (SQ, 1), 0)) // BLK
            mask = (qb == kb) | (kb == 0) | ((qb + kb) % 3 == 0)

            dst = out_ref if r == 0 else psend_buf.at[r - 1]

            def round_h(h, _, r=r, mask=mask, dst=dst):
                oh = (head_ids == h).astype(jnp.float32)
                qh = q_buf[r, h]
                kh = k_ref[0, :, h, :]
                s = lax.dot_general(
                    qh, kh, (((1,), (1,)), ((), ()))) * SCALE
                p = jnp.exp(jnp.where(mask, s, NEG))
                vh = v_ref[0, :, h, :]
                dst[h] = lax.dot_general(p, vh, (((1,), (0,)), ((), ())))
                l_col = jnp.sum(p, axis=1, keepdims=True)
                lst_buf[r] = lst_buf[r] * (1.0 - oh) + l_col * oh
                return _
            lax.fori_loop(0, HQ, round_h, None)

            if r > 0:
                d = rdma(psend_buf.at[r - 1], precv_buf.at[r - 1],
                         p_send.at[r - 1], p_recv.at[r - 1], q_idx)
                d.start()
                in_flight.append(d)
                d = rdma(lst_buf.at[r], lrecv_buf.at[r - 1],
                         l_send.at[r - 1], l_recv.at[r - 1], q_idx)
                d.start()
                in_flight.append(d)

        for k in range(1, N_DEV):
            rdma(precv_buf.at[k - 1], precv_buf.at[k - 1],
                 p_send.at[k - 1], p_recv.at[k - 1], left).wait_recv()
            rdma(lrecv_buf.at[k - 1], lrecv_buf.at[k - 1],
                 l_send.at[k - 1], l_recv.at[k - 1], left).wait_recv()

        l_tot = (lst_buf[0] + lrecv_buf[0]
                 + lrecv_buf[1] + lrecv_buf[2])

        def fin_h(h, _):
            oh = (head_ids == h).astype(jnp.float32)
            l_col = jnp.sum(l_tot * oh, axis=1, keepdims=True)
            out_ref[h] = (out_ref[h] + precv_buf[0, h] + precv_buf[1, h]
                          + precv_buf[2, h]) / l_col
            return _
        lax.fori_loop(0, HQ, fin_h, None)

        for d in in_flight:
            d.wait_send()

    return pl.pallas_call(
        body,
        out_shape=jax.ShapeDtypeStruct((HQ, SQ, DH), jnp.float32),
        in_specs=[pl.BlockSpec(memory_space=pltpu.VMEM)] * 3,
        out_specs=pl.BlockSpec(memory_space=pltpu.VMEM),
        scratch_shapes=[
            pltpu.VMEM((N_DEV, HQ, SQ, DH), jnp.float32),
            pltpu.VMEM((N_DEV - 1, HQ, SQ, DH), jnp.float32),
            pltpu.VMEM((N_DEV - 1, HQ, SQ, DH), jnp.float32),
            pltpu.VMEM((N_DEV, SQ, HQ), jnp.float32),
            pltpu.VMEM((N_DEV - 1, SQ, HQ), jnp.float32),
            pltpu.SemaphoreType.DMA((N_DEV - 1,)),
            pltpu.SemaphoreType.DMA((N_DEV - 1,)),
            pltpu.SemaphoreType.DMA((N_DEV - 1,)),
            pltpu.SemaphoreType.DMA((N_DEV - 1,)),
            pltpu.SemaphoreType.DMA((N_DEV - 1,)),
            pltpu.SemaphoreType.DMA((N_DEV - 1,)),
        ],
        compiler_params=pltpu.CompilerParams(
            collective_id=0, vmem_limit_bytes=100 * 1024 * 1024),
    )(q_in, k_in, v_in)


def kernel(x, Wq, K_ext, V_ext, Wo):
    q = (x[0] @ Wq).reshape(SQ, HQ, DH).transpose(1, 0, 2)
    ctx = _attn_ring(q, K_ext, V_ext)
    out = ctx.transpose(1, 0, 2).reshape(SQ, HQ * DH) @ Wo
    return out[None]


# device time: 160858 ns/iter; 2.0970x vs baseline; 1.1667x over previous
import jax
import jax.numpy as jnp
from jax import lax
from jax.experimental import pallas as pl
from jax.experimental.pallas import tpu as pltpu

N_DEV = 4
HQ = 8
DH = 128
SQ = 256
SKV = 4096
BLK = 64
SCALE = 0.08838834764831843
NEG = -1e9


def _attn_ring(q_in, k_in, v_in):

    def body(q_ref, k_ref, v_ref, out_ref,
             q_buf, psend_buf, precv_buf, lst_buf, lrecv_buf,
             q_send, q_recv, p_send, p_recv, l_send, l_recv):
        me = lax.axis_index("i")
        left = (me + N_DEV - 1) % N_DEV
        right = (me + 1) % N_DEV

        barrier = pltpu.get_barrier_semaphore()
        for k in range(1, N_DEV):
            pl.semaphore_signal(barrier, inc=1,
                                device_id=((me + k) % N_DEV,),
                                device_id_type=pl.DeviceIdType.MESH)
        pl.semaphore_wait(barrier, N_DEV - 1)

        q_buf[0] = q_ref[...]

        kb = (me * SKV
              + lax.broadcasted_iota(jnp.int32, (1, SKV), 1)) // BLK
        head_ids = lax.broadcasted_iota(jnp.int32, (1, HQ), 1)

        in_flight = []

        def rdma(src, dst, ssem, rsem, target):
            return pltpu.make_async_remote_copy(
                src_ref=src, dst_ref=dst, send_sem=ssem, recv_sem=rsem,
                device_id=(target,), device_id_type=pl.DeviceIdType.MESH)

        for r in range(N_DEV):
            if r > 0:
                rdma(q_buf.at[r], q_buf.at[r],
                     q_send.at[r - 1], q_recv.at[r - 1], left).wait_recv()
            if r < N_DEV - 1:
                d = rdma(q_buf.at[r], q_buf.at[r + 1],
                         q_send.at[r], q_recv.at[r], right)
                d.start()
                in_flight.append(d)

            q_idx = (me + N_DEV - r) % N_DEV
            qb = (q_idx * SQ
                  + lax.broadcasted_iota(jnp.int32, (SQ, 1), 0)) // BLK
            sum3 = qb % 3 + kb % 3
            mask = (qb == kb) | (kb == 0) | (sum3 == 0) | (sum3 == 3)

            dst = out_ref if r == 0 else psend_buf.at[r - 1]

            def round_h(h, _, r=r, mask=mask, dst=dst, q_idx=q_idx):
                oh = (head_ids == h).astype(jnp.float32)
                qh = q_buf[r, h]
                kh = k_ref[0, :, h, :]
                s = lax.dot_general(
                    qh, kh, (((1,), (1,)), ((), ())),
                    precision=lax.Precision.DEFAULT) * SCALE
                p = jnp.exp(jnp.where(mask, s, NEG))
                vh = v_ref[0, :, h, :]
                dst[h] = lax.dot_general(p, vh, (((1,), (0,)), ((), ())),
                                         precision=lax.Precision.DEFAULT)
                l_col = jnp.sum(p, axis=1, keepdims=True)
                lst_buf[r] = lst_buf[r] * (1.0 - oh) + l_col * oh
                if r > 0:
                    rdma(psend_buf.at[r - 1, h], precv_buf.at[r - 1, h],
                         p_send.at[r - 1, h], p_recv.at[r - 1, h],
                         q_idx).start()
                return _
            lax.fori_loop(0, HQ, round_h, None)

            if r > 0:
                d = rdma(lst_buf.at[r], lrecv_buf.at[r - 1],
                         l_send.at[r - 1], l_recv.at[r - 1], q_idx)
                d.start()
                in_flight.append(d)

        for k in range(1, N_DEV):
            rdma(lrecv_buf.at[k - 1], lrecv_buf.at[k - 1],
                 l_send.at[k - 1], l_recv.at[k - 1], left).wait_recv()

        l_tot = (lst_buf[0] + lrecv_buf[0]
                 + lrecv_buf[1] + lrecv_buf[2])

        def fin_h(h, _):
            for k in range(1, N_DEV):
                rdma(precv_buf.at[k - 1, h], precv_buf.at[k - 1, h],
                     p_recv.at[k - 1, h], p_recv.at[k - 1, h],
                     left).wait_recv()
            oh = (head_ids == h).astype(jnp.float32)
            l_col = jnp.sum(l_tot * oh, axis=1, keepdims=True)
            out_ref[h] = (out_ref[h] + precv_buf[0, h] + precv_buf[1, h]
                          + precv_buf[2, h]) / l_col
            return _
        lax.fori_loop(0, HQ, fin_h, None)

        def drain_h(h, _):
            for r in range(1, N_DEV):
                rdma(psend_buf.at[r - 1, h], psend_buf.at[r - 1, h],
                     p_send.at[r - 1, h], p_send.at[r - 1, h],
                     left).wait_send()
            return _
        lax.fori_loop(0, HQ, drain_h, None)

        for d in in_flight:
            d.wait_send()

    return pl.pallas_call(
        body,
        out_shape=jax.ShapeDtypeStruct((HQ, SQ, DH), jnp.float32),
        in_specs=[pl.BlockSpec(memory_space=pltpu.VMEM)] * 3,
        out_specs=pl.BlockSpec(memory_space=pltpu.VMEM),
        scratch_shapes=[
            pltpu.VMEM((N_DEV, HQ, SQ, DH), jnp.float32),
            pltpu.VMEM((N_DEV - 1, HQ, SQ, DH), jnp.float32),
            pltpu.VMEM((N_DEV - 1, HQ, SQ, DH), jnp.float32),
            pltpu.VMEM((N_DEV, SQ, HQ), jnp.float32),
            pltpu.VMEM((N_DEV - 1, SQ, HQ), jnp.float32),
            pltpu.SemaphoreType.DMA((N_DEV - 1,)),
            pltpu.SemaphoreType.DMA((N_DEV - 1,)),
            pltpu.SemaphoreType.DMA((N_DEV - 1, HQ)),
            pltpu.SemaphoreType.DMA((N_DEV - 1, HQ)),
            pltpu.SemaphoreType.DMA((N_DEV - 1,)),
            pltpu.SemaphoreType.DMA((N_DEV - 1,)),
        ],
        compiler_params=pltpu.CompilerParams(
            collective_id=0, vmem_limit_bytes=100 * 1024 * 1024),
    )(q_in, k_in, v_in)


def kernel(x, Wq, K_ext, V_ext, Wo):
    q = (x[0] @ Wq).reshape(SQ, HQ, DH).transpose(1, 0, 2)
    ctx = _attn_ring(q, K_ext, V_ext)
    out = ctx.transpose(1, 0, 2).reshape(SQ, HQ * DH) @ Wo
    return out[None]
